# baseline (speedup 1.0000x reference)
import functools

import jax
import jax.numpy as jnp
from jax import lax
from jax.experimental import pallas as pl
from jax.experimental.pallas import tpu as pltpu
from jax.experimental.pallas import tpu_sc as plsc

D = 64
NC = 2
NS = 16
NW = NC * NS
GROUP = 512
SUB = 128
K = GROUP // SUB
NBUF = 2


@functools.lru_cache(maxsize=None)
def _build(B):
    assert B % (NW * GROUP * NBUF) == 0
    bpw = B // NW
    ng = bpw // GROUP
    mesh = plsc.VectorSubcoreMesh(core_axis_name="c", subcore_axis_name="s")

    @functools.partial(
        pl.kernel,
        out_type=jax.ShapeDtypeStruct((B, D), jnp.float32),
        mesh=mesh,
        scratch_types=[
            pltpu.VMEM((NBUF, GROUP), jnp.int32),
            pltpu.VMEM((NBUF, GROUP, D), jnp.float32),
            pltpu.SemaphoreType.DMA,
            pltpu.SemaphoreType.DMA,
        ],
        compiler_params=pltpu.CompilerParams(use_tc_tiling_on_sc=False),
    )
    def emb(idx_hbm, table_hbm, out_hbm, idx_v, rows_v, sem0, sem1):
        wid = lax.axis_index("s") * NC + lax.axis_index("c")
        base = wid * bpw
        sems = [sem0, sem1]

        def fire(g, slot):
            pltpu.sync_copy(idx_hbm.at[pl.ds(base + g * GROUP, GROUP)],
                            idx_v.at[slot])
            for j in range(K):
                pltpu.async_copy(
                    table_hbm.at[idx_v.at[slot, pl.ds(j * SUB, SUB)]],
                    rows_v.at[slot, pl.ds(j * SUB, SUB)],
                    sems[slot])

        def drain_write(g, slot):
            for j in range(K):
                pltpu.make_async_copy(
                    table_hbm.at[idx_v.at[slot, pl.ds(j * SUB, SUB)]],
                    rows_v.at[slot, pl.ds(j * SUB, SUB)],
                    sems[slot]).wait()
            pltpu.sync_copy(rows_v.at[slot],
                            out_hbm.at[pl.ds(base + g * GROUP, GROUP)])

        fire(0, 0)
        fire(1, 1)

        def step(t, carry):
            g0 = t * 2
            drain_write(g0, 0)

            @pl.when(g0 + 2 < ng)
            def _():
                fire(g0 + 2, 0)

            drain_write(g0 + 1, 1)

            @pl.when(g0 + 3 < ng)
            def _():
                fire(g0 + 3, 1)

            return carry

        lax.fori_loop(0, ng // 2, step, 0)

    return emb


def kernel(x, table):
    b, h = x.shape
    flat = x.reshape(b * h).astype(jnp.int32)
    out = _build(b * h)(flat, table)
    return out.reshape(b, h, D)


# speedup vs baseline: 1.0085x; 1.0085x over previous
import functools

import jax
import jax.numpy as jnp
from jax import lax
from jax.experimental import pallas as pl
from jax.experimental.pallas import tpu as pltpu
from jax.experimental.pallas import tpu_sc as plsc

D = 64
NC = 2
NS = 16
NW = NC * NS
NBUF = 4


@functools.lru_cache(maxsize=None)
def _build(B, H):
    rpw = B // NW
    assert rpw % NBUF == 0
    mesh = plsc.VectorSubcoreMesh(core_axis_name="c", subcore_axis_name="s")

    @functools.partial(
        pl.kernel,
        out_type=jax.ShapeDtypeStruct((B, H, D), jnp.float32),
        mesh=mesh,
        scratch_types=[
            pltpu.VMEM((rpw, H), jnp.int32),
            pltpu.VMEM((NBUF, H, D), jnp.float32),
            pltpu.SemaphoreType.DMA,
            pltpu.SemaphoreType.DMA,
            pltpu.SemaphoreType.DMA,
            pltpu.SemaphoreType.DMA,
        ],
        compiler_params=pltpu.CompilerParams(use_tc_tiling_on_sc=False),
    )
    def emb(x_hbm, table_hbm, out_hbm, idx_v, rows_v, s0, s1, s2, s3):
        wid = lax.axis_index("s") * NC + lax.axis_index("c")
        base = wid * rpw
        sems = [s0, s1, s2, s3]

        pltpu.sync_copy(x_hbm.at[pl.ds(base, rpw), :], idx_v)

        def fire(r, slot):
            pltpu.async_copy(table_hbm.at[idx_v.at[r]], rows_v.at[slot],
                             sems[slot])

        def drain_write(r, slot):
            pltpu.make_async_copy(table_hbm.at[idx_v.at[r]],
                                  rows_v.at[slot], sems[slot]).wait()
            pltpu.sync_copy(rows_v.at[slot], out_hbm.at[base + r])

        for b in range(NBUF):
            fire(b, b)

        def step(t, carry):
            r0 = t * NBUF
            for b in range(NBUF):
                drain_write(r0 + b, b)

                @pl.when(r0 + b + NBUF < rpw)
                def _():
                    fire(r0 + b + NBUF, b)

            return carry

        lax.fori_loop(0, rpw // NBUF, step, 0)

    return emb


def kernel(x, table):
    b, h = x.shape
    return _build(b, h)(x.astype(jnp.int32), table)


# speedup vs baseline: 1.3408x; 1.3295x over previous
import functools

import jax
import jax.numpy as jnp
from jax import lax
from jax.experimental import pallas as pl
from jax.experimental.pallas import tpu as pltpu
from jax.experimental.pallas import tpu_sc as plsc

D = 64
NC = 2
NS = 16
NW = NC * NS
NBUF = 4


@functools.lru_cache(maxsize=None)
def _build(B, H):
    rpw = B // NW
    assert rpw % NBUF == 0
    mesh = plsc.VectorSubcoreMesh(core_axis_name="c", subcore_axis_name="s")

    @functools.partial(
        pl.kernel,
        out_type=jax.ShapeDtypeStruct((B, H, 2 * D), jnp.float32),
        mesh=mesh,
        scratch_types=[
            pltpu.VMEM((rpw, H), jnp.int32),
            pltpu.VMEM((NBUF, H, D), jnp.float32),
            pltpu.SemaphoreType.DMA,
            pltpu.SemaphoreType.DMA,
            pltpu.SemaphoreType.DMA,
            pltpu.SemaphoreType.DMA,
        ],
        compiler_params=pltpu.CompilerParams(use_tc_tiling_on_sc=False),
    )
    def emb(x_hbm, table_hbm, out_hbm, idx_v, rows_v, s0, s1, s2, s3):
        wid = lax.axis_index("s") * NC + lax.axis_index("c")
        base = wid * rpw
        sems = [s0, s1, s2, s3]

        pltpu.sync_copy(x_hbm.at[pl.ds(base, rpw), :], idx_v)

        def fire(r, slot):
            pltpu.async_copy(table_hbm.at[idx_v.at[r]], rows_v.at[slot],
                             sems[slot])

        def drain_write(r, slot):
            pltpu.make_async_copy(table_hbm.at[idx_v.at[r]],
                                  rows_v.at[slot], sems[slot]).wait()
            pltpu.sync_copy(rows_v.at[slot],
                            out_hbm.at[base + r, :, pl.ds(0, D)])

        for b in range(NBUF):
            fire(b, b)

        def step(t, carry):
            r0 = t * NBUF
            for b in range(NBUF):
                drain_write(r0 + b, b)

                @pl.when(r0 + b + NBUF < rpw)
                def _():
                    fire(r0 + b + NBUF, b)

            return carry

        lax.fori_loop(0, rpw // NBUF, step, 0)

    def kern(x, table):
        out = emb(x.astype(jnp.int32), table)
        return out[:, :, :D]

    return kern


def kernel(x, table):
    b, h = x.shape
    return _build(b, h)(x, table)
